# baseline (speedup 1.0000x reference)
import functools

import jax
import jax.numpy as jnp
from jax import lax
from jax.experimental import pallas as pl
from jax.experimental.pallas import tpu as pltpu
from jax.experimental.pallas import tpu_sc as plsc

N_PAD = 100352
E_ROWS = 25088
E_PAD = E_ROWS * 128
NB = 16
STRIPE = N_PAD // 16

_mesh = plsc.VectorSubcoreMesh(core_axis_name="c", subcore_axis_name="s")
_sc_params = pltpu.CompilerParams(use_tc_tiling_on_sc=False)


@functools.partial(
    pl.kernel,
    out_type=jax.ShapeDtypeStruct((2, N_PAD), jnp.float32),
    mesh=_mesh,
    compiler_params=_sc_params,
    scratch_types=[
        pltpu.VMEM_SHARED((N_PAD,), jnp.float32),
        pltpu.VMEM((NB, 128), jnp.int32),
        pltpu.VMEM((128,), jnp.float32),
    ],
)
def _deg_pass(dst_hbm, ones_hbm, zeros_hbm, out_hbm, acc, dstb, ones_v):
    c = lax.axis_index("c")
    s = lax.axis_index("s")
    off = s * STRIPE
    pltpu.sync_copy(zeros_hbm.at[pl.ds(off, STRIPE)], acc.at[pl.ds(off, STRIPE)])
    pltpu.sync_copy(ones_hbm, ones_v)
    plsc.subcore_barrier()
    tile_rows = E_ROWS // 32
    row0 = (c * 16 + s) * tile_rows

    def blk(b, carry):
        r0 = row0 + b * NB
        pltpu.sync_copy(dst_hbm.at[pl.ds(r0, NB)], dstb)
        for j in range(NB):
            pltpu.sync_copy(ones_v, acc.at[dstb.at[j]], add=True)
        return carry

    lax.fori_loop(0, tile_rows // NB, blk, 0)
    plsc.subcore_barrier()
    pltpu.sync_copy(acc.at[pl.ds(off, STRIPE)], out_hbm.at[c, pl.ds(off, STRIPE)])


def _make_gather_pass(width, dual, nb):
    NB = nb
    vec = width > 1
    tabshape = (N_PAD, width) if vec else (N_PAD,)
    vshape = (NB, 128, width) if vec else (NB, 128)
    n_tab = 2 if dual else 1
    tile_rows = E_ROWS // (16 if dual else 32)
    nblocks = tile_rows // NB

    def body(*refs):
        src_hbm, dst_hbm = refs[0], refs[1]
        tabs = refs[2:2 + n_tab]
        (zeros_hbm, out_hbm, acc, srcb, dstb, vals,
         semg, sems0, sems1) = refs[2 + n_tab:]
        sems = (sems0, sems1)
        c = lax.axis_index("c")
        s = lax.axis_index("s")
        off = s * STRIPE
        pltpu.sync_copy(zeros_hbm.at[pl.ds(off, STRIPE)],
                        acc.at[pl.ds(off, STRIPE)])
        plsc.subcore_barrier()

        def scat_descs(bank):
            return [
                pltpu.make_async_copy(vals.at[bank, j],
                                      acc.at[dstb.at[bank, j]], sems[bank])
                for j in range(NB)
            ]

        def run(tab, row0):
            def do_block(b, bank, guarded):
                r0 = row0 + b * NB
                if guarded:
                    @pl.when(b >= 2)
                    def _():
                        for d in scat_descs(bank):
                            d.wait()
                else:
                    for d in scat_descs(bank):
                        d.wait()
                pltpu.sync_copy(src_hbm.at[pl.ds(r0, NB)], srcb.at[bank])
                pltpu.sync_copy(dst_hbm.at[pl.ds(r0, NB)], dstb.at[bank])
                gds = [
                    pltpu.make_async_copy(tab.at[srcb.at[bank, j]],
                                          vals.at[bank, j], semg)
                    for j in range(NB)
                ]
                for d in gds:
                    d.start()
                for d in gds:
                    d.wait()
                for d in scat_descs(bank):
                    d.start(add=True)

            def sup(hb, carry):
                do_block(2 * hb, 0, True)
                do_block(2 * hb + 1, 1, True)
                return carry

            lax.fori_loop(0, nblocks // 2, sup, 0)
            for bank in range(2):
                for d in scat_descs(bank):
                    d.wait()

        if dual:
            @pl.when(c == 0)
            def _():
                run(tabs[0], s * tile_rows)

            @pl.when(c == 1)
            def _():
                run(tabs[1], s * tile_rows)
        else:
            run(tabs[0], (c * 16 + s) * tile_rows)

        plsc.subcore_barrier()
        pltpu.sync_copy(acc.at[pl.ds(off, STRIPE)],
                        out_hbm.at[c, pl.ds(off, STRIPE)])

    return pl.kernel(
        body,
        out_type=jax.ShapeDtypeStruct((2,) + tabshape, jnp.float32),
        mesh=_mesh,
        compiler_params=_sc_params,
        scratch_types=[
            pltpu.VMEM_SHARED(tabshape, jnp.float32),
            pltpu.VMEM((2, NB, 128), jnp.int32),
            pltpu.VMEM((2, NB, 128), jnp.int32),
            pltpu.VMEM((2,) + vshape, jnp.float32),
            pltpu.SemaphoreType.DMA,
            pltpu.SemaphoreType.DMA,
            pltpu.SemaphoreType.DMA,
        ],
    )


_spmv1 = _make_gather_pass(1, dual=False, nb=8)
_spmm16 = _make_gather_pass(16, dual=False, nb=4)
_spmm_dual = _make_gather_pass(16, dual=True, nb=4)


BN = 2048
GRID = N_PAD // BN

_vspec = pl.BlockSpec((BN,), lambda i: (i,))
_fspec = pl.BlockSpec((BN, 16), lambda i: (i, 0))


def _prep_body(d0, d1, x, dinv, q):
    deg = d0[...] + d1[...] + 1.0
    r = 1.0 / jnp.sqrt(deg)
    dinv[...] = r
    q[...] = x[...] * r


_prep = pl.pallas_call(
    _prep_body,
    grid=(GRID,),
    in_specs=[_vspec] * 3,
    out_specs=[_vspec] * 2,
    out_shape=[jax.ShapeDtypeStruct((N_PAD,), jnp.float32)] * 2,
)


def _dense1_body(s0, s1, q, dinv, w, b, g2):
    dv = dinv[...]
    t1 = dv * (s0[...] + s1[...] + q[...])
    h = jnp.maximum(t1[:, None] * w[...][None, :] + b[...][None, :], 0.0)
    g2[...] = dv[:, None] * h


_dense1 = pl.pallas_call(
    _dense1_body,
    grid=(GRID,),
    in_specs=[_vspec] * 4 + [
        pl.BlockSpec((16,), lambda i: (0,)),
        pl.BlockSpec((16,), lambda i: (0,)),
    ],
    out_specs=_fspec,
    out_shape=jax.ShapeDtypeStruct((N_PAD, 16), jnp.float32),
)


def _dense2_body(s0, s1, g2, dinv, w2, b2, g3a, g3b):
    dv = dinv[...]
    t2 = dv[:, None] * (s0[...] + s1[...] + g2[...])
    w2r = w2[...].astype(jnp.bfloat16).astype(jnp.float32)
    h2 = jnp.dot(t2, w2r, preferred_element_type=jnp.float32, precision=lax.Precision.HIGHEST)
    h2 = jnp.maximum(h2 + b2[...][None, :], 0.0)
    g3 = dv[:, None] * h2
    g3a[...] = g3[:, :16]
    g3b[...] = g3[:, 16:]


_dense2 = pl.pallas_call(
    _dense2_body,
    grid=(GRID,),
    in_specs=[_fspec] * 3 + [
        _vspec,
        pl.BlockSpec((16, 32), lambda i: (0, 0)),
        pl.BlockSpec((32,), lambda i: (0,)),
    ],
    out_specs=[_fspec, _fspec],
    out_shape=[jax.ShapeDtypeStruct((N_PAD, 16), jnp.float32)] * 2,
)


def _pool_body(s3a, s3b, g3a, g3b, dinv, bid, sums, cnts):
    i = pl.program_id(0)
    dv = dinv[...]
    t3 = jnp.concatenate(
        [dv[:, None] * (s3a[...] + g3a[...]),
         dv[:, None] * (s3b[...] + g3b[...])], axis=1)
    seg = lax.broadcasted_iota(jnp.int32, (64, BN), 0)
    oh = (bid[...][None, :] == seg).astype(jnp.float32)
    ps = jnp.dot(oh, t3, preferred_element_type=jnp.float32, precision=lax.Precision.HIGHEST)
    pc = jnp.sum(oh, axis=1)

    @pl.when(i == 0)
    def _():
        sums[...] = jnp.zeros_like(sums)
        cnts[...] = jnp.zeros_like(cnts)

    sums[...] += ps
    cnts[...] += pc


_pool = pl.pallas_call(
    _pool_body,
    grid=(GRID,),
    in_specs=[_fspec] * 4 + [_vspec, _vspec],
    out_specs=[
        pl.BlockSpec((64, 32), lambda i: (0, 0)),
        pl.BlockSpec((64,), lambda i: (0,)),
    ],
    out_shape=[
        jax.ShapeDtypeStruct((64, 32), jnp.float32),
        jax.ShapeDtypeStruct((64,), jnp.float32),
    ],
)


def _head_body(sums, cnts, w3, b3, fw1, fb1, fw2, fb2, out):
    cnt = cnts[...]
    pooled = sums[...] / jnp.maximum(cnt, 1.0)[:, None]
    w3r = w3[...].astype(jnp.bfloat16).astype(jnp.float32)
    p3 = jnp.dot(pooled, w3r, preferred_element_type=jnp.float32, precision=lax.Precision.HIGHEST)
    p3 = p3 + b3[...][None, :]
    p3 = p3 * (cnt > 0.0).astype(jnp.float32)[:, None]
    p3r = p3.astype(jnp.bfloat16).astype(jnp.float32)
    w1r = fw1[...].astype(jnp.bfloat16).astype(jnp.float32)
    z = jnp.dot(p3r, w1r, preferred_element_type=jnp.float32, precision=lax.Precision.HIGHEST)
    z = jnp.maximum(z + fb1[...][None, :], 0.0)
    zr = z.astype(jnp.bfloat16).astype(jnp.float32)
    w2r = fw2[...].astype(jnp.bfloat16).astype(jnp.float32)
    out[...] = (jnp.dot(zr, w2r, preferred_element_type=jnp.float32, precision=lax.Precision.HIGHEST)
                + fb2[...][None, :])


_head = pl.pallas_call(
    _head_body,
    out_shape=jax.ShapeDtypeStruct((64, 2), jnp.float32),
)


def kernel(x, edge_index, batch, W1, b1, W2, b2, W3, b3, fW1, fb1, fW2, fb2):
    n = x.shape[0]
    e = edge_index.shape[1]
    src = edge_index[0].astype(jnp.int32)
    dst = edge_index[1].astype(jnp.int32)
    src2d = jnp.concatenate(
        [src, jnp.zeros((E_PAD - e,), jnp.int32)]).reshape(E_ROWS, 128)
    dst2d = jnp.concatenate(
        [dst, jnp.full((E_PAD - e,), n, jnp.int32)]).reshape(E_ROWS, 128)
    xp = jnp.pad(x[:, 0], (0, N_PAD - n))
    bid = jnp.pad(batch.astype(jnp.int32), (0, N_PAD - n), constant_values=127)
    zeros1 = jnp.zeros((N_PAD,), jnp.float32)
    zeros16 = jnp.zeros((N_PAD, 16), jnp.float32)
    ones128 = jnp.ones((128,), jnp.float32)

    dp = _deg_pass(dst2d, ones128, zeros1)
    dinv, q = _prep(dp[0], dp[1], xp)
    s1 = _spmv1(src2d, dst2d, q, zeros1)
    g2 = _dense1(s1[0], s1[1], q, dinv, W1.reshape(16), b1)
    s2 = _spmm16(src2d, dst2d, g2, zeros16)
    g3a, g3b = _dense2(s2[0], s2[1], g2, dinv, W2, b2)
    s3 = _spmm_dual(src2d, dst2d, g3a, g3b, zeros16)
    sums, cnts = _pool(s3[0], s3[1], g3a, g3b, dinv, bid)
    return _head(sums, cnts, W3, b3, fW1, fb1, fW2, fb2)


# speedup vs baseline: 1.0632x; 1.0632x over previous
import functools

import jax
import jax.numpy as jnp
from jax import lax
from jax.experimental import pallas as pl
from jax.experimental.pallas import tpu as pltpu
from jax.experimental.pallas import tpu_sc as plsc

N_PAD = 100352
E_ROWS = 25344
E_PAD = E_ROWS * 128
DEG_NB = 12
STRIPE = N_PAD // 16

_mesh = plsc.VectorSubcoreMesh(core_axis_name="c", subcore_axis_name="s")
_sc_params = pltpu.CompilerParams(use_tc_tiling_on_sc=False)


@functools.partial(
    pl.kernel,
    out_type=jax.ShapeDtypeStruct((2, N_PAD), jnp.float32),
    mesh=_mesh,
    compiler_params=_sc_params,
    scratch_types=[
        pltpu.VMEM_SHARED((N_PAD,), jnp.float32),
        pltpu.VMEM((2, DEG_NB, 128), jnp.int32),
        pltpu.VMEM((128,), jnp.float32),
        pltpu.SemaphoreType.DMA,
        pltpu.SemaphoreType.DMA,
    ],
)
def _deg_pass(dst_hbm, ones_hbm, zeros_hbm, out_hbm, acc, dstb, ones_v,
              sems0, sems1):
    NB = DEG_NB
    sems = (sems0, sems1)
    c = lax.axis_index("c")
    s = lax.axis_index("s")
    off = s * STRIPE
    pltpu.sync_copy(zeros_hbm.at[pl.ds(off, STRIPE)], acc.at[pl.ds(off, STRIPE)])
    pltpu.sync_copy(ones_hbm, ones_v)
    plsc.subcore_barrier()
    tile_rows = E_ROWS // 32
    row0 = (c * 16 + s) * tile_rows

    def sdescs(bank):
        return [pltpu.make_async_copy(ones_v, acc.at[dstb.at[bank, j]],
                                      sems[bank]) for j in range(NB)]

    def do_block(b, bank):
        @pl.when(b >= 2)
        def _():
            for d in sdescs(bank):
                d.wait()
        pltpu.sync_copy(dst_hbm.at[pl.ds(row0 + b * NB, NB)], dstb.at[bank])
        for d in sdescs(bank):
            d.start(add=True)

    def sup(hb, carry):
        do_block(2 * hb, 0)
        do_block(2 * hb + 1, 1)
        return carry

    lax.fori_loop(0, tile_rows // (2 * NB), sup, 0)
    for bank in range(2):
        for d in sdescs(bank):
            d.wait()
    plsc.subcore_barrier()
    pltpu.sync_copy(acc.at[pl.ds(off, STRIPE)], out_hbm.at[c, pl.ds(off, STRIPE)])


def _make_gather_pass(width, dual, nb):
    NB = nb
    vec = width > 1
    tabshape = (N_PAD, width) if vec else (N_PAD,)
    vshape = (NB, 128, width) if vec else (NB, 128)
    n_tab = 2 if dual else 1
    tile_rows = E_ROWS // (16 if dual else 32)
    nblocks = tile_rows // NB

    def body(*refs):
        src_hbm, dst_hbm = refs[0], refs[1]
        tabs = refs[2:2 + n_tab]
        (zeros_hbm, out_hbm, acc, srcb, dstb, vals,
         semg0, semg1) = refs[2 + n_tab:]
        semg = (semg0, semg1)
        c = lax.axis_index("c")
        s = lax.axis_index("s")
        off = s * STRIPE
        pltpu.sync_copy(zeros_hbm.at[pl.ds(off, STRIPE)],
                        acc.at[pl.ds(off, STRIPE)])
        plsc.subcore_barrier()

        def run(tab, row0):
            def gdescs(bank):
                return [
                    pltpu.make_async_copy(tab.at[srcb.at[bank, j]],
                                          vals.at[bank, j], semg[bank])
                    for j in range(NB)
                ]

            def load_fire(b, bank):
                r0 = row0 + b * NB
                pltpu.sync_copy(src_hbm.at[pl.ds(r0, NB)], srcb.at[bank])
                pltpu.sync_copy(dst_hbm.at[pl.ds(r0, NB)], dstb.at[bank])
                for d in gdescs(bank):
                    d.start()

            def drain(bank):
                for d in gdescs(bank):
                    d.wait()
                for j in range(NB):
                    pltpu.sync_copy(vals.at[bank, j],
                                    acc.at[dstb.at[bank, j]], add=True)

            load_fire(0, 0)

            def sup(hb, carry):
                b0 = 2 * hb
                load_fire(b0 + 1, 1)
                drain(0)

                @pl.when(b0 + 2 < nblocks)
                def _():
                    load_fire(b0 + 2, 0)

                drain(1)
                return carry

            lax.fori_loop(0, nblocks // 2, sup, 0)

        if dual:
            @pl.when(c == 0)
            def _():
                run(tabs[0], s * tile_rows)

            @pl.when(c == 1)
            def _():
                run(tabs[1], s * tile_rows)
        else:
            run(tabs[0], (c * 16 + s) * tile_rows)

        plsc.subcore_barrier()
        pltpu.sync_copy(acc.at[pl.ds(off, STRIPE)],
                        out_hbm.at[c, pl.ds(off, STRIPE)])

    return pl.kernel(
        body,
        out_type=jax.ShapeDtypeStruct((2,) + tabshape, jnp.float32),
        mesh=_mesh,
        compiler_params=_sc_params,
        scratch_types=[
            pltpu.VMEM_SHARED(tabshape, jnp.float32),
            pltpu.VMEM((2, NB, 128), jnp.int32),
            pltpu.VMEM((2, NB, 128), jnp.int32),
            pltpu.VMEM((2,) + vshape, jnp.float32),
            pltpu.SemaphoreType.DMA,
            pltpu.SemaphoreType.DMA,
        ],
    )


_spmv1 = _make_gather_pass(1, dual=False, nb=22)
_spmm16 = _make_gather_pass(16, dual=False, nb=6)
_spmm_dual = _make_gather_pass(16, dual=True, nb=6)


BN = 2048
GRID = N_PAD // BN

_vspec = pl.BlockSpec((BN,), lambda i: (i,))
_fspec = pl.BlockSpec((BN, 16), lambda i: (i, 0))


def _prep_body(d0, d1, x, dinv, q):
    deg = d0[...] + d1[...] + 1.0
    r = 1.0 / jnp.sqrt(deg)
    dinv[...] = r
    q[...] = x[...] * r


_prep = pl.pallas_call(
    _prep_body,
    grid=(GRID,),
    in_specs=[_vspec] * 3,
    out_specs=[_vspec] * 2,
    out_shape=[jax.ShapeDtypeStruct((N_PAD,), jnp.float32)] * 2,
)


def _dense1_body(s0, s1, q, dinv, w, b, g2):
    dv = dinv[...]
    t1 = dv * (s0[...] + s1[...] + q[...])
    h = jnp.maximum(t1[:, None] * w[...][None, :] + b[...][None, :], 0.0)
    g2[...] = dv[:, None] * h


_dense1 = pl.pallas_call(
    _dense1_body,
    grid=(GRID,),
    in_specs=[_vspec] * 4 + [
        pl.BlockSpec((16,), lambda i: (0,)),
        pl.BlockSpec((16,), lambda i: (0,)),
    ],
    out_specs=_fspec,
    out_shape=jax.ShapeDtypeStruct((N_PAD, 16), jnp.float32),
)


def _dense2_body(s0, s1, g2, dinv, w2, b2, g3a, g3b):
    dv = dinv[...]
    t2 = dv[:, None] * (s0[...] + s1[...] + g2[...])
    w2r = w2[...].astype(jnp.bfloat16).astype(jnp.float32)
    h2 = jnp.dot(t2, w2r, preferred_element_type=jnp.float32, precision=lax.Precision.HIGHEST)
    h2 = jnp.maximum(h2 + b2[...][None, :], 0.0)
    g3 = dv[:, None] * h2
    g3a[...] = g3[:, :16]
    g3b[...] = g3[:, 16:]


_dense2 = pl.pallas_call(
    _dense2_body,
    grid=(GRID,),
    in_specs=[_fspec] * 3 + [
        _vspec,
        pl.BlockSpec((16, 32), lambda i: (0, 0)),
        pl.BlockSpec((32,), lambda i: (0,)),
    ],
    out_specs=[_fspec, _fspec],
    out_shape=[jax.ShapeDtypeStruct((N_PAD, 16), jnp.float32)] * 2,
)


def _pool_body(s3a, s3b, g3a, g3b, dinv, bid, sums, cnts):
    i = pl.program_id(0)
    dv = dinv[...]
    t3 = jnp.concatenate(
        [dv[:, None] * (s3a[...] + g3a[...]),
         dv[:, None] * (s3b[...] + g3b[...])], axis=1)
    seg = lax.broadcasted_iota(jnp.int32, (64, BN), 0)
    oh = (bid[...][None, :] == seg).astype(jnp.float32)
    ps = jnp.dot(oh, t3, preferred_element_type=jnp.float32, precision=lax.Precision.HIGHEST)
    pc = jnp.sum(oh, axis=1)

    @pl.when(i == 0)
    def _():
        sums[...] = jnp.zeros_like(sums)
        cnts[...] = jnp.zeros_like(cnts)

    sums[...] += ps
    cnts[...] += pc


_pool = pl.pallas_call(
    _pool_body,
    grid=(GRID,),
    in_specs=[_fspec] * 4 + [_vspec, _vspec],
    out_specs=[
        pl.BlockSpec((64, 32), lambda i: (0, 0)),
        pl.BlockSpec((64,), lambda i: (0,)),
    ],
    out_shape=[
        jax.ShapeDtypeStruct((64, 32), jnp.float32),
        jax.ShapeDtypeStruct((64,), jnp.float32),
    ],
)


def _head_body(sums, cnts, w3, b3, fw1, fb1, fw2, fb2, out):
    cnt = cnts[...]
    pooled = sums[...] / jnp.maximum(cnt, 1.0)[:, None]
    w3r = w3[...].astype(jnp.bfloat16).astype(jnp.float32)
    p3 = jnp.dot(pooled, w3r, preferred_element_type=jnp.float32, precision=lax.Precision.HIGHEST)
    p3 = p3 + b3[...][None, :]
    p3 = p3 * (cnt > 0.0).astype(jnp.float32)[:, None]
    p3r = p3.astype(jnp.bfloat16).astype(jnp.float32)
    w1r = fw1[...].astype(jnp.bfloat16).astype(jnp.float32)
    z = jnp.dot(p3r, w1r, preferred_element_type=jnp.float32, precision=lax.Precision.HIGHEST)
    z = jnp.maximum(z + fb1[...][None, :], 0.0)
    zr = z.astype(jnp.bfloat16).astype(jnp.float32)
    w2r = fw2[...].astype(jnp.bfloat16).astype(jnp.float32)
    out[...] = (jnp.dot(zr, w2r, preferred_element_type=jnp.float32, precision=lax.Precision.HIGHEST)
                + fb2[...][None, :])


_head = pl.pallas_call(
    _head_body,
    out_shape=jax.ShapeDtypeStruct((64, 2), jnp.float32),
)


def kernel(x, edge_index, batch, W1, b1, W2, b2, W3, b3, fW1, fb1, fW2, fb2):
    n = x.shape[0]
    e = edge_index.shape[1]
    src = edge_index[0].astype(jnp.int32)
    dst = edge_index[1].astype(jnp.int32)
    src2d = jnp.concatenate(
        [src, jnp.zeros((E_PAD - e,), jnp.int32)]).reshape(E_ROWS, 128)
    dst2d = jnp.concatenate(
        [dst, jnp.full((E_PAD - e,), n, jnp.int32)]).reshape(E_ROWS, 128)
    xp = jnp.pad(x[:, 0], (0, N_PAD - n))
    bid = jnp.pad(batch.astype(jnp.int32), (0, N_PAD - n), constant_values=127)
    zeros1 = jnp.zeros((N_PAD,), jnp.float32)
    zeros16 = jnp.zeros((N_PAD, 16), jnp.float32)
    ones128 = jnp.ones((128,), jnp.float32)

    dp = _deg_pass(dst2d, ones128, zeros1)
    dinv, q = _prep(dp[0], dp[1], xp)
    s1 = _spmv1(src2d, dst2d, q, zeros1)
    g2 = _dense1(s1[0], s1[1], q, dinv, W1.reshape(16), b1)
    s2 = _spmm16(src2d, dst2d, g2, zeros16)
    g3a, g3b = _dense2(s2[0], s2[1], g2, dinv, W2, b2)
    s3 = _spmm_dual(src2d, dst2d, g3a, g3b, zeros16)
    sums, cnts = _pool(s3[0], s3[1], g3a, g3b, dinv, bid)
    return _head(sums, cnts, W3, b3, fW1, fb1, fW2, fb2)
